# Initial kernel scaffold; baseline (speedup 1.0000x reference)
import functools

import jax
import jax.numpy as jnp
from jax import lax
from jax.experimental import pallas as pl
from jax.experimental.pallas import tpu as pltpu
from jax.experimental.pallas import tpu_sc as plsc

NP = 10240
D = 128
NC = 2
NS = 16
NW = NC * NS
CHUNK = 128
ROWS_PER_TILE = NP // NS
ZCOPIES = ROWS_PER_TILE // CHUNK

BR = 2048
NG = NP // 128
BG = BR // 128


def _sc_mesh():
    return plsc.VectorSubcoreMesh(
        core_axis_name="c", subcore_axis_name="s", num_cores=NC, num_subcores=NS
    )


def _make_deg_kernel(cpt):
    @functools.partial(
        pl.kernel,
        out_type=jax.ShapeDtypeStruct((NC, NP), jnp.float32),
        mesh=_sc_mesh(),
        scratch_types=[
            pltpu.VMEM((cpt, CHUNK), jnp.int32),
            pltpu.VMEM((CHUNK,), jnp.float32),
            pltpu.VMEM((ROWS_PER_TILE,), jnp.float32),
            pltpu.VMEM_SHARED((NP,), jnp.float32),
            pltpu.SemaphoreType.DMA,
        ],
    )
    def deg_kernel(dst_hbm, out_hbm, idx_v, ones_v, zeros_v, hist_sh, sem):
        cid = lax.axis_index("c")
        sid = lax.axis_index("s")
        wid = cid * NS + sid
        for i in range(CHUNK // 16):
            ones_v[pl.ds(i * 16, 16)] = jnp.ones((16,), jnp.float32)
        for i in range(ROWS_PER_TILE // 16):
            zeros_v[pl.ds(i * 16, 16)] = jnp.zeros((16,), jnp.float32)
        pltpu.sync_copy(zeros_v, hist_sh.at[pl.ds(sid * ROWS_PER_TILE, ROWS_PER_TILE)])
        pltpu.sync_copy(dst_hbm.at[wid], idx_v)
        plsc.subcore_barrier()

        def body(j, carry):
            pltpu.async_copy(ones_v, hist_sh.at[idx_v.at[j]], sem, add=True).wait()
            return carry

        lax.fori_loop(0, cpt, body, 0)
        plsc.subcore_barrier()
        pltpu.sync_copy(
            hist_sh.at[pl.ds(sid * ROWS_PER_TILE, ROWS_PER_TILE)],
            out_hbm.at[cid, pl.ds(sid * ROWS_PER_TILE, ROWS_PER_TILE)],
        )

    return deg_kernel


def _make_spmm_kernel(cpt):
    @functools.partial(
        pl.kernel,
        out_type=jax.ShapeDtypeStruct((NC, NP, D), jnp.float32),
        mesh=_sc_mesh(),
        scratch_types=[
            pltpu.VMEM((cpt, CHUNK), jnp.int32),
            pltpu.VMEM((cpt, CHUNK), jnp.int32),
            pltpu.VMEM((CHUNK, D), jnp.float32),
            pltpu.VMEM((CHUNK, D), jnp.float32),
            pltpu.VMEM_SHARED((NP, D), jnp.float32),
            pltpu.SemaphoreType.DMA,
            pltpu.SemaphoreType.DMA,
        ],
    )
    def spmm_kernel(y_hbm, src_hbm, dst_hbm, out_hbm,
                    srcv, dstv, buf0, buf1, acc_sh, sem0, sem1):
        cid = lax.axis_index("c")
        sid = lax.axis_index("s")
        wid = cid * NS + sid
        pltpu.sync_copy(src_hbm.at[wid], srcv)
        pltpu.sync_copy(dst_hbm.at[wid], dstv)

        def zbody(r, carry):
            for i in range(D // 16):
                buf0[r, pl.ds(i * 16, 16)] = jnp.zeros((16,), jnp.float32)
            return carry

        lax.fori_loop(0, CHUNK, zbody, 0)
        for k in range(ZCOPIES):
            pltpu.sync_copy(
                buf0, acc_sh.at[pl.ds(sid * ROWS_PER_TILE + k * CHUNK, CHUNK)]
            )
        plsc.subcore_barrier()

        pltpu.async_copy(y_hbm.at[srcv.at[0]], buf0, sem0)

        def body(i, carry):
            j = 2 * i
            nxt = jnp.minimum(j + 2, cpt - 1)
            pltpu.async_copy(y_hbm.at[srcv.at[j + 1]], buf1, sem1)
            pltpu.make_async_copy(y_hbm.at[srcv.at[j]], buf0, sem0).wait()
            pltpu.sync_copy(buf0, acc_sh.at[dstv.at[j]], add=True)
            pltpu.async_copy(y_hbm.at[srcv.at[nxt]], buf0, sem0)
            pltpu.make_async_copy(y_hbm.at[srcv.at[j + 1]], buf1, sem1).wait()
            pltpu.sync_copy(buf1, acc_sh.at[dstv.at[j + 1]], add=True)
            return carry

        lax.fori_loop(0, cpt // 2, body, 0)
        pltpu.make_async_copy(y_hbm.at[srcv.at[cpt - 1]], buf0, sem0).wait()
        plsc.subcore_barrier()
        for k in range(ZCOPIES):
            pltpu.sync_copy(
                acc_sh.at[pl.ds(sid * ROWS_PER_TILE + k * CHUNK, CHUNK)],
                out_hbm.at[cid, pl.ds(sid * ROWS_PER_TILE + k * CHUNK, CHUNK)],
            )

    return spmm_kernel


def _scale_body(x_ref, w_ref, dp_ref, y_ref):
    dis = lax.rsqrt(1.0 + dp_ref[0] + dp_ref[1])
    xw = jnp.dot(x_ref[...], w_ref[...], preferred_element_type=jnp.float32)
    y_ref[...] = (xw.reshape(BG, 128, D) * dis[:, :, None]).reshape(BR, D)


def _mid_body(ap_ref, y_ref, dp_ref, b_ref, w_ref, o_ref):
    dis = lax.rsqrt(1.0 + dp_ref[0] + dp_ref[1])
    acc = ap_ref[0] + ap_ref[1] + y_ref[...]
    pre = (acc.reshape(BG, 128, D) * dis[:, :, None]).reshape(BR, D) + b_ref[...]
    h = jnp.maximum(pre, 0.0)
    xw = jnp.dot(h, w_ref[...], preferred_element_type=jnp.float32)
    o_ref[...] = (xw.reshape(BG, 128, D) * dis[:, :, None]).reshape(BR, D)


def _out_body(ap_ref, y_ref, dp_ref, b_ref, wc_ref, bc_ref, o_ref):
    dis = lax.rsqrt(1.0 + dp_ref[0] + dp_ref[1])
    acc = ap_ref[0] + ap_ref[1] + y_ref[...]
    pre = (acc.reshape(BG, 128, D) * dis[:, :, None]).reshape(BR, D) + b_ref[...]
    h = jnp.maximum(pre, 0.0)
    o_ref[...] = (
        jnp.dot(h, wc_ref[...], preferred_element_type=jnp.float32) + bc_ref[...]
    )


def _row_spec(width):
    return pl.BlockSpec((BR, width), lambda g: (g, 0))


_DP_SPEC = pl.BlockSpec((2, BG, 128), lambda g: (0, g, 0))
_AP_SPEC = pl.BlockSpec((2, BR, D), lambda g: (0, g, 0))


def _const_spec(shape):
    nd = len(shape)
    return pl.BlockSpec(shape, lambda g: (0,) * nd)


def _tc_scale(xp, W, dp3):
    return pl.pallas_call(
        _scale_body,
        grid=(NP // BR,),
        in_specs=[_row_spec(D), _const_spec((D, D)), _DP_SPEC],
        out_specs=_row_spec(D),
        out_shape=jax.ShapeDtypeStruct((NP, D), jnp.float32),
    )(xp, W, dp3)


def _tc_mid(ap, y, dp3, b, W):
    return pl.pallas_call(
        _mid_body,
        grid=(NP // BR,),
        in_specs=[_AP_SPEC, _row_spec(D), _DP_SPEC, _const_spec((1, D)),
                  _const_spec((D, D))],
        out_specs=_row_spec(D),
        out_shape=jax.ShapeDtypeStruct((NP, D), jnp.float32),
    )(ap, y, dp3, b, W)


def _tc_out(ap, y, dp3, b, Wc, bc):
    ne = Wc.shape[1]
    return pl.pallas_call(
        _out_body,
        grid=(NP // BR,),
        in_specs=[_AP_SPEC, _row_spec(D), _DP_SPEC, _const_spec((1, D)),
                  _const_spec((D, ne)), _const_spec((1, ne))],
        out_specs=_row_spec(ne),
        out_shape=jax.ShapeDtypeStruct((NP, ne), jnp.float32),
    )(ap, y, dp3, b, Wc, bc)


def kernel(x, edge_index, W1, b1, W2, b2, Wc, bc):
    n, _ = x.shape
    e = edge_index.shape[1]
    cpt = -(-e // (NW * CHUNK))
    cpt += cpt % 2
    ep = NW * CHUNK * cpt

    src = edge_index[0].astype(jnp.int32)
    dst = edge_index[1].astype(jnp.int32)
    src3 = jnp.concatenate([src, jnp.zeros((ep - e,), jnp.int32)]).reshape(
        NW, cpt, CHUNK
    )
    dst3 = jnp.concatenate([dst, jnp.full((ep - e,), n, jnp.int32)]).reshape(
        NW, cpt, CHUNK
    )
    xp = jnp.pad(x, ((0, NP - n), (0, 0)))
    b1r = b1.reshape(1, D)
    b2r = b2.reshape(1, D)
    bcr = bc.reshape(1, -1)

    degparts = _make_deg_kernel(cpt)(dst3)
    dp3 = degparts.reshape(NC, NG, 128)

    spmm = _make_spmm_kernel(cpt)
    y1 = _tc_scale(xp, W1, dp3)
    ap1 = spmm(y1, src3, dst3)
    y2 = _tc_mid(ap1, y1, dp3, b1r, W2)
    ap2 = spmm(y2, src3, dst3)
    out = _tc_out(ap2, y2, dp3, b2r, Wc, bcr)
    return out[:n]


# speedup vs baseline: 9.3338x; 9.3338x over previous
import functools

import jax
import jax.numpy as jnp
from jax import lax
from jax.experimental import pallas as pl
from jax.experimental.pallas import tpu as pltpu
from jax.experimental.pallas import tpu_sc as plsc

NP = 10240
D = 128
NC = 2
NS = 16
NW = NC * NS
CHUNK = 128
ROWS_PER_TILE = NP // NS
ZCOPIES = ROWS_PER_TILE // CHUNK

BR = 2048
NG = NP // 128
BG = BR // 128


def _sc_mesh():
    return plsc.VectorSubcoreMesh(
        core_axis_name="c", subcore_axis_name="s", num_cores=NC, num_subcores=NS
    )


def _make_deg_kernel(cpt):
    @functools.partial(
        pl.kernel,
        out_type=jax.ShapeDtypeStruct((NC, NP), jnp.float32),
        mesh=_sc_mesh(),
        scratch_types=[
            pltpu.VMEM((cpt, CHUNK), jnp.int32),
            pltpu.VMEM((CHUNK,), jnp.float32),
            pltpu.VMEM((ROWS_PER_TILE,), jnp.float32),
            pltpu.VMEM_SHARED((NP,), jnp.float32),
            pltpu.SemaphoreType.DMA,
        ],
    )
    def deg_kernel(dst_hbm, out_hbm, idx_v, ones_v, zeros_v, hist_sh, sem):
        cid = lax.axis_index("c")
        sid = lax.axis_index("s")
        wid = cid * NS + sid
        for i in range(CHUNK // 16):
            ones_v[pl.ds(i * 16, 16)] = jnp.ones((16,), jnp.float32)
        for i in range(ROWS_PER_TILE // 16):
            zeros_v[pl.ds(i * 16, 16)] = jnp.zeros((16,), jnp.float32)
        pltpu.sync_copy(zeros_v, hist_sh.at[pl.ds(sid * ROWS_PER_TILE, ROWS_PER_TILE)])
        pltpu.sync_copy(dst_hbm.at[wid], idx_v)
        plsc.subcore_barrier()

        def body(j, carry):
            pltpu.async_copy(ones_v, hist_sh.at[idx_v.at[j]], sem, add=True).wait()
            return carry

        lax.fori_loop(0, cpt, body, 0)
        plsc.subcore_barrier()
        pltpu.sync_copy(
            hist_sh.at[pl.ds(sid * ROWS_PER_TILE, ROWS_PER_TILE)],
            out_hbm.at[cid, pl.ds(sid * ROWS_PER_TILE, ROWS_PER_TILE)],
        )

    return deg_kernel


def _make_spmm_kernel(cpt):
    @functools.partial(
        pl.kernel,
        out_type=jax.ShapeDtypeStruct((NC, NP, D), jnp.float32),
        mesh=_sc_mesh(),
        scratch_types=[
            pltpu.VMEM((cpt, CHUNK), jnp.int32),
            pltpu.VMEM((cpt, CHUNK), jnp.int32),
            pltpu.VMEM((CHUNK, D), jnp.float32),
            pltpu.VMEM_SHARED((NP, D), jnp.float32),
            pltpu.SemaphoreType.DMA,
        ],
    )
    def spmm_kernel(y_hbm, src_hbm, dst_hbm, out_hbm,
                    srcv, dstv, buf0, acc_sh, sem0):
        cid = lax.axis_index("c")
        sid = lax.axis_index("s")
        wid = cid * NS + sid
        pltpu.sync_copy(src_hbm.at[wid], srcv)
        pltpu.sync_copy(dst_hbm.at[wid], dstv)

        def zbody(r, carry):
            for i in range(D // 16):
                buf0[r, pl.ds(i * 16, 16)] = jnp.zeros((16,), jnp.float32)
            return carry

        lax.fori_loop(0, CHUNK, zbody, 0)
        for k in range(ZCOPIES):
            pltpu.sync_copy(
                buf0, acc_sh.at[pl.ds(sid * ROWS_PER_TILE + k * CHUNK, CHUNK)]
            )
        plsc.subcore_barrier()

        def body(j, carry):
            pltpu.async_copy(y_hbm.at[srcv.at[j]], buf0, sem0).wait()
            pltpu.sync_copy(buf0, acc_sh.at[dstv.at[j]], add=True)
            return carry

        lax.fori_loop(0, cpt, body, 0)
        plsc.subcore_barrier()
        for k in range(ZCOPIES):
            pltpu.sync_copy(
                acc_sh.at[pl.ds(sid * ROWS_PER_TILE + k * CHUNK, CHUNK)],
                out_hbm.at[cid, pl.ds(sid * ROWS_PER_TILE + k * CHUNK, CHUNK)],
            )

    return spmm_kernel


def _scale_body(x_ref, w_ref, dp_ref, y_ref):
    dis = lax.rsqrt(1.0 + dp_ref[0] + dp_ref[1])
    xw = jnp.dot(x_ref[...], w_ref[...], preferred_element_type=jnp.float32)
    y_ref[...] = (xw.reshape(BG, 128, D) * dis[:, :, None]).reshape(BR, D)


def _mid_body(ap_ref, y_ref, dp_ref, b_ref, w_ref, o_ref):
    dis = lax.rsqrt(1.0 + dp_ref[0] + dp_ref[1])
    acc = ap_ref[0] + ap_ref[1] + y_ref[...]
    pre = (acc.reshape(BG, 128, D) * dis[:, :, None]).reshape(BR, D) + b_ref[...]
    h = jnp.maximum(pre, 0.0)
    xw = jnp.dot(h, w_ref[...], preferred_element_type=jnp.float32)
    o_ref[...] = (xw.reshape(BG, 128, D) * dis[:, :, None]).reshape(BR, D)


def _out_body(ap_ref, y_ref, dp_ref, b_ref, wc_ref, bc_ref, o_ref):
    dis = lax.rsqrt(1.0 + dp_ref[0] + dp_ref[1])
    acc = ap_ref[0] + ap_ref[1] + y_ref[...]
    pre = (acc.reshape(BG, 128, D) * dis[:, :, None]).reshape(BR, D) + b_ref[...]
    h = jnp.maximum(pre, 0.0)
    o_ref[...] = (
        jnp.dot(h, wc_ref[...], preferred_element_type=jnp.float32) + bc_ref[...]
    )


def _row_spec(width):
    return pl.BlockSpec((BR, width), lambda g: (g, 0))


_DP_SPEC = pl.BlockSpec((2, BG, 128), lambda g: (0, g, 0))
_AP_SPEC = pl.BlockSpec((2, BR, D), lambda g: (0, g, 0))


def _const_spec(shape):
    nd = len(shape)
    return pl.BlockSpec(shape, lambda g: (0,) * nd)


def _tc_scale(xp, W, dp3):
    return pl.pallas_call(
        _scale_body,
        grid=(NP // BR,),
        in_specs=[_row_spec(D), _const_spec((D, D)), _DP_SPEC],
        out_specs=_row_spec(D),
        out_shape=jax.ShapeDtypeStruct((NP, D), jnp.float32),
    )(xp, W, dp3)


def _tc_mid(ap, y, dp3, b, W):
    return pl.pallas_call(
        _mid_body,
        grid=(NP // BR,),
        in_specs=[_AP_SPEC, _row_spec(D), _DP_SPEC, _const_spec((1, D)),
                  _const_spec((D, D))],
        out_specs=_row_spec(D),
        out_shape=jax.ShapeDtypeStruct((NP, D), jnp.float32),
    )(ap, y, dp3, b, W)


def _tc_out(ap, y, dp3, b, Wc, bc):
    ne = Wc.shape[1]
    return pl.pallas_call(
        _out_body,
        grid=(NP // BR,),
        in_specs=[_AP_SPEC, _row_spec(D), _DP_SPEC, _const_spec((1, D)),
                  _const_spec((D, ne)), _const_spec((1, ne))],
        out_specs=_row_spec(ne),
        out_shape=jax.ShapeDtypeStruct((NP, ne), jnp.float32),
    )(ap, y, dp3, b, Wc, bc)


def kernel(x, edge_index, W1, b1, W2, b2, Wc, bc):
    n, _ = x.shape
    e = edge_index.shape[1]
    cpt = -(-e // (NW * CHUNK))
    cpt += cpt % 2
    ep = NW * CHUNK * cpt

    src = edge_index[0].astype(jnp.int32)
    dst = edge_index[1].astype(jnp.int32)
    src3 = jnp.concatenate([src, jnp.zeros((ep - e,), jnp.int32)]).reshape(
        NW, cpt, CHUNK
    )
    dst3 = jnp.concatenate([dst, jnp.full((ep - e,), n, jnp.int32)]).reshape(
        NW, cpt, CHUNK
    )
    xp = jnp.pad(x, ((0, NP - n), (0, 0)))
    b1r = b1.reshape(1, D)
    b2r = b2.reshape(1, D)
    bcr = bc.reshape(1, -1)

    degparts = _make_deg_kernel(cpt)(dst3)
    dp3 = degparts.reshape(NC, NG, 128)

    spmm = _make_spmm_kernel(cpt)
    y1 = _tc_scale(xp, W1, dp3)
    ap1 = spmm(y1, src3, dst3)
    y2 = _tc_mid(ap1, y1, dp3, b1r, W2)
    ap2 = spmm(y2, src3, dst3)
    out = _tc_out(ap2, y2, dp3, b2r, Wc, bcr)
    return out[:n]


# speedup vs baseline: 10.4135x; 1.1157x over previous
import functools

import jax
import jax.numpy as jnp
from jax import lax
from jax.experimental import pallas as pl
from jax.experimental.pallas import tpu as pltpu
from jax.experimental.pallas import tpu_sc as plsc

NP = 10240
D = 128
NC = 2
NS = 16
NW = NC * NS
CHUNK = 128
ROWS_PER_TILE = NP // NS
ZCOPIES = ROWS_PER_TILE // CHUNK

BR = 2048
NG = NP // 128
BG = BR // 128


def _sc_mesh():
    return plsc.VectorSubcoreMesh(
        core_axis_name="c", subcore_axis_name="s", num_cores=NC, num_subcores=NS
    )


def _make_deg_kernel(cpt):
    @functools.partial(
        pl.kernel,
        out_type=jax.ShapeDtypeStruct((NC, NP), jnp.float32),
        mesh=_sc_mesh(),
        scratch_types=[
            pltpu.VMEM((cpt, CHUNK), jnp.int32),
            pltpu.VMEM((CHUNK,), jnp.float32),
            pltpu.VMEM((ROWS_PER_TILE,), jnp.float32),
            pltpu.VMEM_SHARED((NP,), jnp.float32),
            pltpu.SemaphoreType.DMA,
        ],
    )
    def deg_kernel(dst_hbm, out_hbm, idx_v, ones_v, zeros_v, hist_sh, sem):
        cid = lax.axis_index("c")
        sid = lax.axis_index("s")
        wid = cid * NS + sid
        for i in range(CHUNK // 16):
            ones_v[pl.ds(i * 16, 16)] = jnp.ones((16,), jnp.float32)
        for i in range(ROWS_PER_TILE // 16):
            zeros_v[pl.ds(i * 16, 16)] = jnp.zeros((16,), jnp.float32)
        pltpu.sync_copy(zeros_v, hist_sh.at[pl.ds(sid * ROWS_PER_TILE, ROWS_PER_TILE)])
        pltpu.sync_copy(dst_hbm.at[wid], idx_v)
        plsc.subcore_barrier()

        def body(j, carry):
            pltpu.async_copy(ones_v, hist_sh.at[idx_v.at[j]], sem, add=True).wait()
            return carry

        lax.fori_loop(0, cpt, body, 0)
        plsc.subcore_barrier()
        pltpu.sync_copy(
            hist_sh.at[pl.ds(sid * ROWS_PER_TILE, ROWS_PER_TILE)],
            out_hbm.at[cid, pl.ds(sid * ROWS_PER_TILE, ROWS_PER_TILE)],
        )

    return deg_kernel


SEG = 16


def _make_spmm_kernel(cpt):
    nseg = cpt // SEG

    @functools.partial(
        pl.kernel,
        out_type=jax.ShapeDtypeStruct((NC, NP, D), jnp.float32),
        mesh=_sc_mesh(),
        scratch_types=[
            pltpu.VMEM((2, SEG, CHUNK), jnp.int32),
            pltpu.VMEM((2, SEG, CHUNK), jnp.int32),
            pltpu.VMEM((CHUNK, D), jnp.float32),
            pltpu.VMEM((CHUNK, D), jnp.float32),
            pltpu.VMEM_SHARED((NP, D), jnp.float32),
            pltpu.SemaphoreType.DMA,
            pltpu.SemaphoreType.DMA,
            pltpu.SemaphoreType.DMA,
        ],
    )
    def spmm_kernel(y_hbm, src_hbm, dst_hbm, out_hbm,
                    srcv, dstv, buf0, buf1, acc_sh, gsem0, gsem1, isem):
        cid = lax.axis_index("c")
        sid = lax.axis_index("s")
        wid = cid * NS + sid
        pltpu.sync_copy(src_hbm.at[wid, 0], srcv.at[0])
        pltpu.sync_copy(dst_hbm.at[wid, 0], dstv.at[0])

        def zbody(r, carry):
            for i in range(D // 16):
                buf0[r, pl.ds(i * 16, 16)] = jnp.zeros((16,), jnp.float32)
            return carry

        lax.fori_loop(0, CHUNK, zbody, 0)
        for k in range(ZCOPIES):
            pltpu.sync_copy(
                buf0, acc_sh.at[pl.ds(sid * ROWS_PER_TILE + k * CHUNK, CHUNK)]
            )
        plsc.subcore_barrier()

        def seg_body(s, carry):
            p = lax.rem(s, 2)
            q = lax.rem(s + 1, 2)

            @pl.when(s < nseg - 1)
            def _prefetch():
                pltpu.async_copy(src_hbm.at[wid, s + 1], srcv.at[q], isem)
                pltpu.async_copy(dst_hbm.at[wid, s + 1], dstv.at[q], isem)

            pltpu.async_copy(y_hbm.at[srcv.at[p, 0]], buf0, gsem0)

            def pair(i, c):
                j = 2 * i
                pltpu.async_copy(y_hbm.at[srcv.at[p, j + 1]], buf1, gsem1)
                pltpu.make_async_copy(y_hbm.at[srcv.at[p, j]], buf0, gsem0).wait()
                pltpu.sync_copy(buf0, acc_sh.at[dstv.at[p, j]], add=True)

                @pl.when(j + 2 < SEG)
                def _next():
                    pltpu.async_copy(y_hbm.at[srcv.at[p, j + 2]], buf0, gsem0)

                pltpu.make_async_copy(
                    y_hbm.at[srcv.at[p, j + 1]], buf1, gsem1
                ).wait()
                pltpu.sync_copy(buf1, acc_sh.at[dstv.at[p, j + 1]], add=True)
                return c

            lax.fori_loop(0, SEG // 2, pair, 0)

            @pl.when(s < nseg - 1)
            def _drain_idx():
                pltpu.make_async_copy(src_hbm.at[wid, s + 1], srcv.at[q], isem).wait()
                pltpu.make_async_copy(dst_hbm.at[wid, s + 1], dstv.at[q], isem).wait()

            return carry

        lax.fori_loop(0, nseg, seg_body, 0)
        plsc.subcore_barrier()
        for k in range(ZCOPIES):
            pltpu.sync_copy(
                acc_sh.at[pl.ds(sid * ROWS_PER_TILE + k * CHUNK, CHUNK)],
                out_hbm.at[cid, pl.ds(sid * ROWS_PER_TILE + k * CHUNK, CHUNK)],
            )

    return spmm_kernel


def _scale_body(x_ref, w_ref, dp_ref, y_ref):
    dis = lax.rsqrt(1.0 + dp_ref[0] + dp_ref[1])
    xw = jnp.dot(x_ref[...], w_ref[...], preferred_element_type=jnp.float32)
    y_ref[...] = (xw.reshape(BG, 128, D) * dis[:, :, None]).reshape(BR, D)


def _mid_body(ap_ref, y_ref, dp_ref, b_ref, w_ref, o_ref):
    dis = lax.rsqrt(1.0 + dp_ref[0] + dp_ref[1])
    acc = ap_ref[0] + ap_ref[1] + y_ref[...]
    pre = (acc.reshape(BG, 128, D) * dis[:, :, None]).reshape(BR, D) + b_ref[...]
    h = jnp.maximum(pre, 0.0)
    xw = jnp.dot(h, w_ref[...], preferred_element_type=jnp.float32)
    o_ref[...] = (xw.reshape(BG, 128, D) * dis[:, :, None]).reshape(BR, D)


def _out_body(ap_ref, y_ref, dp_ref, b_ref, wc_ref, bc_ref, o_ref):
    dis = lax.rsqrt(1.0 + dp_ref[0] + dp_ref[1])
    acc = ap_ref[0] + ap_ref[1] + y_ref[...]
    pre = (acc.reshape(BG, 128, D) * dis[:, :, None]).reshape(BR, D) + b_ref[...]
    h = jnp.maximum(pre, 0.0)
    o_ref[...] = (
        jnp.dot(h, wc_ref[...], preferred_element_type=jnp.float32) + bc_ref[...]
    )


def _row_spec(width):
    return pl.BlockSpec((BR, width), lambda g: (g, 0))


_DP_SPEC = pl.BlockSpec((2, BG, 128), lambda g: (0, g, 0))
_AP_SPEC = pl.BlockSpec((2, BR, D), lambda g: (0, g, 0))


def _const_spec(shape):
    nd = len(shape)
    return pl.BlockSpec(shape, lambda g: (0,) * nd)


def _tc_scale(xp, W, dp3):
    return pl.pallas_call(
        _scale_body,
        grid=(NP // BR,),
        in_specs=[_row_spec(D), _const_spec((D, D)), _DP_SPEC],
        out_specs=_row_spec(D),
        out_shape=jax.ShapeDtypeStruct((NP, D), jnp.float32),
    )(xp, W, dp3)


def _tc_mid(ap, y, dp3, b, W):
    return pl.pallas_call(
        _mid_body,
        grid=(NP // BR,),
        in_specs=[_AP_SPEC, _row_spec(D), _DP_SPEC, _const_spec((1, D)),
                  _const_spec((D, D))],
        out_specs=_row_spec(D),
        out_shape=jax.ShapeDtypeStruct((NP, D), jnp.float32),
    )(ap, y, dp3, b, W)


def _tc_out(ap, y, dp3, b, Wc, bc):
    ne = Wc.shape[1]
    return pl.pallas_call(
        _out_body,
        grid=(NP // BR,),
        in_specs=[_AP_SPEC, _row_spec(D), _DP_SPEC, _const_spec((1, D)),
                  _const_spec((D, ne)), _const_spec((1, ne))],
        out_specs=_row_spec(ne),
        out_shape=jax.ShapeDtypeStruct((NP, ne), jnp.float32),
    )(ap, y, dp3, b, Wc, bc)


def kernel(x, edge_index, W1, b1, W2, b2, Wc, bc):
    n, _ = x.shape
    e = edge_index.shape[1]
    cpt = -(-e // (NW * CHUNK))
    cpt = -(-cpt // SEG) * SEG
    ep = NW * CHUNK * cpt

    src = edge_index[0].astype(jnp.int32)
    dst = edge_index[1].astype(jnp.int32)
    src3 = jnp.concatenate([src, jnp.zeros((ep - e,), jnp.int32)]).reshape(
        NW, cpt, CHUNK
    )
    dst3 = jnp.concatenate([dst, jnp.full((ep - e,), n, jnp.int32)]).reshape(
        NW, cpt, CHUNK
    )
    src4 = src3.reshape(NW, cpt // SEG, SEG, CHUNK)
    dst4 = dst3.reshape(NW, cpt // SEG, SEG, CHUNK)
    xp = jnp.pad(x, ((0, NP - n), (0, 0)))
    b1r = b1.reshape(1, D)
    b2r = b2.reshape(1, D)
    bcr = bc.reshape(1, -1)

    degparts = _make_deg_kernel(cpt)(dst3)
    dp3 = degparts.reshape(NC, NG, 128)

    spmm = _make_spmm_kernel(cpt)
    y1 = _tc_scale(xp, W1, dp3)
    ap1 = spmm(y1, src4, dst4)
    y2 = _tc_mid(ap1, y1, dp3, b1r, W2)
    ap2 = spmm(y2, src4, dst4)
    out = _tc_out(ap2, y2, dp3, b2r, Wc, bcr)
    return out[:n]
